# Initial kernel scaffold; baseline (speedup 1.0000x reference)
import jax
import jax.numpy as jnp
from jax.experimental import pallas as pl


def kernel(x, emb, W1, b1, W2, b2, W3, b3):
    raise NotImplementedError("write your pallas kernel here")


# speedup vs baseline: 4.1765x; 4.1765x over previous
import functools

import jax
import jax.numpy as jnp
from jax import lax
from jax.experimental import pallas as pl
from jax.experimental.pallas import tpu as pltpu
from jax.experimental.pallas import tpu_sc as plsc


def _gather_body(num_chunks, chunk, per_w, nc, idx_hbm, table_hbm, out_hbm,
                 idx_v, rows_v, sem):
    wid = lax.axis_index("s") * nc + lax.axis_index("c")
    base = wid * per_w

    def step(i, carry):
        off = base + i * chunk
        pltpu.sync_copy(idx_hbm.at[pl.ds(off, chunk)], idx_v)
        pltpu.async_copy(table_hbm.at[idx_v], rows_v, sem).wait()
        pltpu.sync_copy(rows_v, out_hbm.at[pl.ds(off, chunk)])
        return carry

    lax.fori_loop(0, num_chunks, step, 0)


def _sc_gather(idx_flat, table):
    total = idx_flat.shape[0]
    emb = table.shape[1]
    info = plsc.get_sparse_core_info()
    nc, ns = info.num_cores, info.num_subcores
    nw = nc * ns
    per_w = total // nw
    chunk = 1024
    num_chunks = per_w // chunk
    mesh = plsc.VectorSubcoreMesh(core_axis_name="c", subcore_axis_name="s")

    kern = functools.partial(
        pl.kernel,
        mesh=mesh,
        compiler_params=pltpu.CompilerParams(use_tc_tiling_on_sc=False),
        out_type=jax.ShapeDtypeStruct((total, emb), jnp.float32),
        scratch_types=[
            pltpu.VMEM((chunk,), jnp.int32),
            pltpu.VMEM((chunk, emb), jnp.float32),
            pltpu.SemaphoreType.DMA,
        ],
    )(functools.partial(_gather_body, num_chunks, chunk, per_w, nc))
    return kern(idx_flat, table)


def _mlp_body(g_ref, w1_ref, b1_ref, w2_ref, b2_ref, w3_ref, b3_ref, o_ref):
    w23 = jnp.dot(w2_ref[...], w3_ref[...], preferred_element_type=jnp.float32)
    weff = jnp.dot(w1_ref[...], w23, preferred_element_type=jnp.float32)
    beff = (jnp.dot(b1_ref[...], w23, preferred_element_type=jnp.float32)
            + jnp.dot(b2_ref[...], w3_ref[...],
                      preferred_element_type=jnp.float32)
            + b3_ref[...])
    h = jnp.dot(g_ref[...], weff, preferred_element_type=jnp.float32) + beff
    o_ref[...] = jax.nn.sigmoid(h)


def _tc_mlp(g, w1, b1, w2, b2, w3, b3):
    b_total, k = g.shape
    cat = w3.shape[1]
    blk = 2048
    grid = (b_total // blk,)
    full = lambda shape: pl.BlockSpec(shape, lambda i: (0, 0))
    return pl.pallas_call(
        _mlp_body,
        grid=grid,
        in_specs=[
            pl.BlockSpec((blk, k), lambda i: (i, 0)),
            full(w1.shape), full((1, b1.shape[0])),
            full(w2.shape), full((1, b2.shape[0])),
            full(w3.shape), full((1, b3.shape[0])),
        ],
        out_specs=pl.BlockSpec((blk, cat), lambda i: (i, 0)),
        out_shape=jax.ShapeDtypeStruct((b_total, cat), jnp.float32),
    )(g, w1, b1.reshape(1, -1), w2, b2.reshape(1, -1), w3, b3.reshape(1, -1))


def kernel(x, emb, W1, b1, W2, b2, W3, b3):
    b, seq = x.shape
    idx_flat = x.astype(jnp.int32).reshape(-1)
    rows = _sc_gather(idx_flat, emb)
    g = rows.reshape(b, seq * emb.shape[1])
    return _tc_mlp(g, W1, b1, W2, b2, W3, b3)


# speedup vs baseline: 14.9782x; 3.5863x over previous
import functools

import jax
import jax.numpy as jnp
from jax import lax
from jax.experimental import pallas as pl
from jax.experimental.pallas import tpu as pltpu
from jax.experimental.pallas import tpu_sc as plsc

_GRP = 4
_ROW = 2 * _GRP


def _fold_body(w1t_ref, w2_ref, w3_ref, b1_ref, b2_ref, b3_ref,
               wr_ref, beff_ref):
    w23 = jnp.dot(w2_ref[...], w3_ref[...], preferred_element_type=jnp.float32)
    wr_ref[...] = jnp.dot(w1t_ref[...], w23,
                          preferred_element_type=jnp.float32)
    beff_ref[...] = (
        jnp.dot(b1_ref[...], w23, preferred_element_type=jnp.float32)
        + jnp.dot(b2_ref[...], w3_ref[...], preferred_element_type=jnp.float32)
        + b3_ref[...])


def _fold_weights(w1t, w2, w3, b1, b2, b3):
    k, cat = w1t.shape[0], w3.shape[1]
    return pl.pallas_call(
        _fold_body,
        out_shape=(jax.ShapeDtypeStruct((k, cat), jnp.float32),
                   jax.ShapeDtypeStruct((1, cat), jnp.float32)),
    )(w1t, w2, w3, b1.reshape(1, -1), b2.reshape(1, -1), b3.reshape(1, -1))


def _ptab_body(emb_ref, t_ref, p_ref):
    p_ref[...] = jnp.dot(emb_ref[...], t_ref[...],
                         preferred_element_type=jnp.float32)


def _pair_table(emb, t):
    v, e = emb.shape
    n = t.shape[1]
    vb = 10000
    return pl.pallas_call(
        _ptab_body,
        grid=(v // vb,),
        in_specs=[pl.BlockSpec((vb, e), lambda i: (i, 0)),
                  pl.BlockSpec((e, n), lambda i: (0, 0))],
        out_specs=pl.BlockSpec((vb, n), lambda i: (i, 0)),
        out_shape=jax.ShapeDtypeStruct((v, n), jnp.float32),
    )(emb, t)


def _idx_body(ngrp, x_ref, o_ref):
    blk, seq = o_ref.shape
    s = lax.broadcasted_iota(jnp.int32, (blk, seq), 1)
    o_ref[...] = x_ref[...].astype(jnp.int32) * ngrp + s // _GRP


def _make_idx(x, ngrp):
    b, seq = x.shape
    blk = 2048
    return pl.pallas_call(
        functools.partial(_idx_body, ngrp),
        grid=(b // blk,),
        in_specs=[pl.BlockSpec((blk, seq), lambda i: (i, 0))],
        out_specs=pl.BlockSpec((blk, seq), lambda i: (i, 0)),
        out_shape=jax.ShapeDtypeStruct((b, seq), jnp.int32),
    )(x)


def _gather_body(num_chunks, chunk, per_w, nc, idx_hbm, table_hbm, out_hbm,
                 idx_v, rows_v, sem):
    wid = lax.axis_index("s") * nc + lax.axis_index("c")
    base = wid * per_w

    def step(i, carry):
        off = base + i * chunk
        pltpu.sync_copy(idx_hbm.at[pl.ds(off, chunk)], idx_v)
        pltpu.async_copy(table_hbm.at[idx_v], rows_v, sem).wait()
        pltpu.sync_copy(rows_v, out_hbm.at[pl.ds(off, chunk)])
        return carry

    lax.fori_loop(0, num_chunks, step, 0)


def _sc_gather(idx_flat, table):
    total = idx_flat.shape[0]
    row = table.shape[1]
    info = plsc.get_sparse_core_info()
    nc, ns = info.num_cores, info.num_subcores
    per_w = total // (nc * ns)
    chunk = 6400
    mesh = plsc.VectorSubcoreMesh(core_axis_name="c", subcore_axis_name="s")

    kern = functools.partial(
        pl.kernel,
        mesh=mesh,
        compiler_params=pltpu.CompilerParams(use_tc_tiling_on_sc=False),
        out_type=jax.ShapeDtypeStruct((total, row), jnp.float32),
        scratch_types=[
            pltpu.VMEM((chunk,), jnp.int32),
            pltpu.VMEM((chunk, row), jnp.float32),
            pltpu.SemaphoreType.DMA,
        ],
    )(functools.partial(_gather_body, per_w // chunk, chunk, per_w, nc))
    return kern(idx_flat, table)


def _out_body(cat, r_ref, beff_ref, o_ref):
    n = r_ref.shape[1]
    q = lax.broadcasted_iota(jnp.int32, (n, cat), 0)
    col = lax.broadcasted_iota(jnp.int32, (n, cat), 1)
    s, j = q // _ROW, q % _ROW
    sel = (j == 2 * (s % _GRP) + col).astype(jnp.float32)
    h = jnp.dot(r_ref[...], sel, preferred_element_type=jnp.float32)
    o_ref[...] = jax.nn.sigmoid(h + beff_ref[...])


def _reduce_out(r2, beff):
    b, n = r2.shape
    cat = beff.shape[1]
    blk = 2048
    return pl.pallas_call(
        functools.partial(_out_body, cat),
        grid=(b // blk,),
        in_specs=[pl.BlockSpec((blk, n), lambda i: (i, 0)),
                  pl.BlockSpec((1, cat), lambda i: (0, 0))],
        out_specs=pl.BlockSpec((blk, cat), lambda i: (i, 0)),
        out_shape=jax.ShapeDtypeStruct((b, cat), jnp.float32),
    )(r2, beff)


def kernel(x, emb, W1, b1, W2, b2, W3, b3):
    b, seq = x.shape
    v, e = emb.shape
    cat = W3.shape[1]
    ngrp = (seq + _GRP - 1) // _GRP
    w1t = W1.reshape(seq, e, W1.shape[1]).transpose(1, 0, 2).reshape(
        e * seq, W1.shape[1])
    wr, beff = _fold_weights(w1t, W2, W3, b1, b2, b3)
    wr3 = wr.reshape(e, seq, cat)
    wr3 = jnp.pad(wr3, ((0, 0), (0, ngrp * _GRP - seq), (0, 0)))
    t104 = wr3.reshape(e, ngrp * _ROW)
    p = _pair_table(emb, t104)
    p13 = p.reshape(v * ngrp, _ROW)
    idx = _make_idx(x, ngrp).reshape(-1)
    r = _sc_gather(idx, p13)
    return _reduce_out(r.reshape(b, seq * _ROW), beff)
